# baseline (speedup 1.0000x reference)
import functools
import jax
import jax.numpy as jnp
from jax import lax
from jax.experimental import pallas as pl
from jax.experimental.pallas import tpu as pltpu, tpu_sc as plsc

TOP_K = 8
N_EXPERTS = 64
N_GROUP = 8
PER_GROUP = N_EXPERTS // N_GROUP
TOPK_GROUP = 4
SCALE = 2.5

TB = 512
NEG = float("-inf")
LANES = 16

NUM_CORES = 2
NUM_SUBCORES = 16
N_SLICES = 4


def _score_body(x_ref, w_ref, b_ref, sfc_ref):
    x = x_ref[...]
    w = w_ref[...]
    logits = jax.lax.dot_general(
        x, w, (((1,), (1,)), ((), ())),
        preferred_element_type=jnp.float32)
    sig = jax.nn.sigmoid(logits.T)
    sfc_ref[...] = sig + b_ref[...]


def _scores(hs, weight, bias, ts, off):
    t, h = hs.shape
    blk0 = off // TB
    return pl.pallas_call(
        _score_body,
        grid=(ts // TB,),
        in_specs=[
            pl.BlockSpec((TB, h), lambda i: (blk0 + i, 0)),
            pl.BlockSpec((N_EXPERTS, h), lambda i: (0, 0)),
            pl.BlockSpec((N_EXPERTS, 1), lambda i: (0, 0)),
        ],
        out_specs=pl.BlockSpec((N_EXPERTS, TB), lambda i: (0, i)),
        out_shape=jax.ShapeDtypeStruct((N_EXPERTS, ts), jnp.float32),
    )(hs, weight, bias)


def _routing_sc(sfc_t, bias_b, t):
    nw = NUM_CORES * NUM_SUBCORES
    tpw = t // nw
    nchunk = tpw // LANES
    mesh = plsc.VectorSubcoreMesh(core_axis_name="c", subcore_axis_name="s")

    @functools.partial(
        pl.kernel,
        mesh=mesh,
        compiler_params=pltpu.CompilerParams(needs_layout_passes=False),
        out_type=[
            jax.ShapeDtypeStruct((t * TOP_K,), jnp.int32),
            jax.ShapeDtypeStruct((t * TOP_K,), jnp.float32),
        ],
        scratch_types=[
            pltpu.VMEM((N_EXPERTS * tpw,), jnp.float32),
            pltpu.VMEM((N_EXPERTS * LANES,), jnp.float32),
            pltpu.VMEM((N_EXPERTS * LANES,), jnp.float32),
            pltpu.VMEM((tpw * TOP_K,), jnp.int32),
            pltpu.VMEM((tpw * TOP_K,), jnp.float32),
            pltpu.SemaphoreType.DMA,
        ],
    )
    def route(sfc_hbm, bias_hbm, idx_hbm, wgt_hbm,
              sfc_v, bias_v, tmp_v, idxo_v, wgto_v, sem):
        wid = lax.axis_index("s") * NUM_CORES + lax.axis_index("c")
        base = wid * tpw
        copies = [
            pltpu.async_copy(sfc_hbm.at[e, pl.ds(base, tpw)],
                             sfc_v.at[pl.ds(e * tpw, tpw)], sem)
            for e in range(N_EXPERTS)
        ]
        pltpu.sync_copy(bias_hbm, bias_v)
        for cp in copies:
            cp.wait()

        lane = lax.iota(jnp.int32, LANES)

        def chunk_body(c, carry):
            off = c * LANES

            def ld(ref, e):
                return ref[pl.ds(e * tpw + off, LANES)]

            gs = []
            gm1 = []
            ga1 = []
            for g in range(N_GROUP):
                v = [ld(sfc_v, g * PER_GROUP + j) for j in range(PER_GROUP)]
                tmp_v[pl.ds(g * PER_GROUP * LANES, LANES)] = v[0]
                m1 = v[0]
                m2 = jnp.full((LANES,), NEG, jnp.float32)
                a1 = jnp.full((LANES,), g * PER_GROUP, jnp.int32)
                for j in range(1, PER_GROUP):
                    tmp_v[pl.ds((g * PER_GROUP + j) * LANES, LANES)] = v[j]
                    gt = v[j] > m1
                    m2 = jnp.maximum(m2, jnp.minimum(m1, v[j]))
                    m1 = jnp.where(gt, v[j], m1)
                    a1 = jnp.where(gt, g * PER_GROUP + j, a1)
                gs.append(m1 + m2)
                gm1.append(m1)
                ga1.append(a1)

            gm = [jnp.zeros((LANES,), jnp.bool_) for _ in range(N_GROUP)]
            for _ in range(TOPK_GROUP):
                m = gs[0]
                for g in range(1, N_GROUP):
                    m = jnp.maximum(m, gs[g])
                am = jnp.full((LANES,), N_GROUP, jnp.int32)
                for g in range(N_GROUP - 1, -1, -1):
                    am = jnp.where(gs[g] == m, g, am)
                for g in range(N_GROUP):
                    sel = am == g
                    gm[g] = jnp.logical_or(gm[g], sel)
                    gs[g] = jnp.where(sel, NEG, gs[g])

            neg_vec = jnp.full((LANES,), NEG, jnp.float32)
            mg = [jnp.where(gm[g], gm1[g], NEG) for g in range(N_GROUP)]
            ag = list(ga1)
            wcols = []
            icols = []
            for k in range(TOP_K):
                m = mg[0]
                for g in range(1, N_GROUP):
                    m = jnp.maximum(m, mg[g])
                wg = jnp.full((LANES,), N_GROUP, jnp.int32)
                for g in range(N_GROUP - 1, -1, -1):
                    wg = jnp.where(mg[g] == m, g, wg)
                ae = ag[0]
                for g in range(1, N_GROUP):
                    ae = jnp.where(wg == g, ag[g], ae)
                wcols.append(m - plsc.load_gather(bias_v, [ae * LANES + lane]))
                icols.append(ae)
                if k == TOP_K - 1:
                    break
                plsc.store_scatter(tmp_v, [ae * LANES + lane], neg_vec)
                gbase = wg * (PER_GROUP * LANES) + lane
                nm = plsc.load_gather(tmp_v, [gbase])
                na = wg * PER_GROUP
                for j in range(1, PER_GROUP):
                    vj = plsc.load_gather(tmp_v, [gbase + j * LANES])
                    gt = vj > nm
                    nm = jnp.where(gt, vj, nm)
                    na = jnp.where(gt, wg * PER_GROUP + j, na)
                for g in range(N_GROUP):
                    selg = wg == g
                    mg[g] = jnp.where(selg, nm, mg[g])
                    ag[g] = jnp.where(selg, na, ag[g])
            den = wcols[0]
            for k in range(1, TOP_K):
                den = den + wcols[k]
            den = den + 1e-20
            tokk = (off + lane) * TOP_K
            for k in range(TOP_K):
                plsc.store_scatter(idxo_v, [tokk + k], icols[k])
                plsc.store_scatter(wgto_v, [tokk + k], wcols[k] / den * SCALE)
            return carry

        lax.fori_loop(0, nchunk, chunk_body, None)
        pltpu.sync_copy(idxo_v, idx_hbm.at[pl.ds(base * TOP_K, tpw * TOP_K)])
        pltpu.sync_copy(wgto_v, wgt_hbm.at[pl.ds(base * TOP_K, tpw * TOP_K)])

    return route(sfc_t, bias_b)


def kernel(hidden_states, weight, e_score_correction_bias):
    bsz, seq_len, h = hidden_states.shape
    t = bsz * seq_len
    hs = hidden_states.reshape(t, h)
    bias = e_score_correction_bias.reshape(N_EXPERTS, 1)
    bias_b = jnp.broadcast_to(bias, (N_EXPERTS, LANES)).reshape(-1)
    sizes = [4096, 8192, 4096] if t == 16384 else [t // N_SLICES] * N_SLICES
    idx_parts = []
    wgt_parts = []
    off = 0
    for ts in sizes:
        sfc_s = _scores(hs, weight, bias, ts, off)
        idx_s, wgt_s = _routing_sc(sfc_s, bias_b, ts)
        idx_parts.append(idx_s)
        wgt_parts.append(wgt_s)
        off += ts
    idx_flat = jnp.concatenate(idx_parts)
    wgt_flat = jnp.concatenate(wgt_parts)
    return idx_flat.reshape(t, TOP_K), wgt_flat.reshape(t, TOP_K)


# speedup vs baseline: 1.0736x; 1.0736x over previous
import functools
import jax
import jax.numpy as jnp
from jax import lax
from jax.experimental import pallas as pl
from jax.experimental.pallas import tpu as pltpu, tpu_sc as plsc

TOP_K = 8
N_EXPERTS = 64
N_GROUP = 8
PER_GROUP = N_EXPERTS // N_GROUP
TOPK_GROUP = 4
SCALE = 2.5

TB = 512
NEG = float("-inf")
LANES = 16

NUM_CORES = 2
NUM_SUBCORES = 16
N_SLICES = 4


def _score_body(x_ref, w_ref, b_ref, sfc_ref):
    x = x_ref[...]
    w = w_ref[...]
    logits = jax.lax.dot_general(
        x, w, (((1,), (1,)), ((), ())),
        preferred_element_type=jnp.float32)
    sig = jax.nn.sigmoid(logits.T)
    sfc_ref[...] = sig + b_ref[...]


def _scores(hs, weight, bias, ts, s):
    t, h = hs.shape
    blk0 = s * (ts // TB)
    return pl.pallas_call(
        _score_body,
        grid=(ts // TB,),
        in_specs=[
            pl.BlockSpec((TB, h), lambda i: (blk0 + i, 0)),
            pl.BlockSpec((N_EXPERTS, h), lambda i: (0, 0)),
            pl.BlockSpec((N_EXPERTS, 1), lambda i: (0, 0)),
        ],
        out_specs=pl.BlockSpec((N_EXPERTS, TB), lambda i: (0, i)),
        out_shape=jax.ShapeDtypeStruct((N_EXPERTS, ts), jnp.float32),
    )(hs, weight, bias)


def _routing_sc(sfc_t, bias_b, t):
    nw = NUM_CORES * NUM_SUBCORES
    tpw = t // nw
    nchunk = tpw // LANES
    mesh = plsc.VectorSubcoreMesh(core_axis_name="c", subcore_axis_name="s")

    @functools.partial(
        pl.kernel,
        mesh=mesh,
        compiler_params=pltpu.CompilerParams(needs_layout_passes=False),
        out_type=[
            jax.ShapeDtypeStruct((t * TOP_K,), jnp.int32),
            jax.ShapeDtypeStruct((t * TOP_K,), jnp.float32),
        ],
        scratch_types=[
            pltpu.VMEM((N_EXPERTS * tpw,), jnp.float32),
            pltpu.VMEM((N_EXPERTS * LANES,), jnp.float32),
            pltpu.VMEM((N_EXPERTS * LANES,), jnp.float32),
            pltpu.VMEM((tpw * TOP_K,), jnp.int32),
            pltpu.VMEM((tpw * TOP_K,), jnp.float32),
            pltpu.SemaphoreType.DMA,
        ],
    )
    def route(sfc_hbm, bias_hbm, idx_hbm, wgt_hbm,
              sfc_v, bias_v, tmp_v, idxo_v, wgto_v, sem):
        wid = lax.axis_index("s") * NUM_CORES + lax.axis_index("c")
        base = wid * tpw
        copies = [
            pltpu.async_copy(sfc_hbm.at[e, pl.ds(base, tpw)],
                             sfc_v.at[pl.ds(e * tpw, tpw)], sem)
            for e in range(N_EXPERTS)
        ]
        pltpu.sync_copy(bias_hbm, bias_v)
        for cp in copies:
            cp.wait()

        lane = lax.iota(jnp.int32, LANES)

        def chunk_body(c, carry):
            off = c * LANES

            def ld(ref, e):
                return ref[pl.ds(e * tpw + off, LANES)]

            gs = []
            gm1 = []
            ga1 = []
            for g in range(N_GROUP):
                v = [ld(sfc_v, g * PER_GROUP + j) for j in range(PER_GROUP)]
                tmp_v[pl.ds(g * PER_GROUP * LANES, LANES)] = v[0]
                m1 = v[0]
                m2 = jnp.full((LANES,), NEG, jnp.float32)
                a1 = jnp.full((LANES,), g * PER_GROUP, jnp.int32)
                for j in range(1, PER_GROUP):
                    tmp_v[pl.ds((g * PER_GROUP + j) * LANES, LANES)] = v[j]
                    gt = v[j] > m1
                    m2 = jnp.maximum(m2, jnp.minimum(m1, v[j]))
                    m1 = jnp.where(gt, v[j], m1)
                    a1 = jnp.where(gt, g * PER_GROUP + j, a1)
                gs.append(m1 + m2)
                gm1.append(m1)
                ga1.append(a1)

            gm = [jnp.zeros((LANES,), jnp.bool_) for _ in range(N_GROUP)]
            for _ in range(TOPK_GROUP):
                m = gs[0]
                for g in range(1, N_GROUP):
                    m = jnp.maximum(m, gs[g])
                am = jnp.full((LANES,), N_GROUP, jnp.int32)
                for g in range(N_GROUP - 1, -1, -1):
                    am = jnp.where(gs[g] == m, g, am)
                for g in range(N_GROUP):
                    sel = am == g
                    gm[g] = jnp.logical_or(gm[g], sel)
                    gs[g] = jnp.where(sel, NEG, gs[g])

            neg_vec = jnp.full((LANES,), NEG, jnp.float32)
            mg = [jnp.where(gm[g], gm1[g], NEG) for g in range(N_GROUP)]
            ag = list(ga1)
            wcols = []
            icols = []
            for k in range(TOP_K):
                m = mg[0]
                for g in range(1, N_GROUP):
                    m = jnp.maximum(m, mg[g])
                wg = jnp.full((LANES,), N_GROUP, jnp.int32)
                for g in range(N_GROUP - 1, -1, -1):
                    wg = jnp.where(mg[g] == m, g, wg)
                ae = ag[0]
                for g in range(1, N_GROUP):
                    ae = jnp.where(wg == g, ag[g], ae)
                wcols.append(m - plsc.load_gather(bias_v, [ae * LANES + lane]))
                icols.append(ae)
                if k == TOP_K - 1:
                    break
                plsc.store_scatter(tmp_v, [ae * LANES + lane], neg_vec)
                gbase = wg * (PER_GROUP * LANES) + lane
                nm = plsc.load_gather(tmp_v, [gbase])
                na = wg * PER_GROUP
                for j in range(1, PER_GROUP):
                    vj = plsc.load_gather(tmp_v, [gbase + j * LANES])
                    gt = vj > nm
                    nm = jnp.where(gt, vj, nm)
                    na = jnp.where(gt, wg * PER_GROUP + j, na)
                for g in range(N_GROUP):
                    selg = wg == g
                    mg[g] = jnp.where(selg, nm, mg[g])
                    ag[g] = jnp.where(selg, na, ag[g])
            den = wcols[0]
            for k in range(1, TOP_K):
                den = den + wcols[k]
            den = den + 1e-20
            tokk = (off + lane) * TOP_K
            for k in range(TOP_K):
                plsc.store_scatter(idxo_v, [tokk + k], icols[k])
                plsc.store_scatter(wgto_v, [tokk + k], wcols[k] / den * SCALE)
            return carry

        lax.fori_loop(0, nchunk, chunk_body, None)
        pltpu.sync_copy(idxo_v, idx_hbm.at[pl.ds(base * TOP_K, tpw * TOP_K)])
        pltpu.sync_copy(wgto_v, wgt_hbm.at[pl.ds(base * TOP_K, tpw * TOP_K)])

    return route(sfc_t, bias_b)


def kernel(hidden_states, weight, e_score_correction_bias):
    bsz, seq_len, h = hidden_states.shape
    t = bsz * seq_len
    hs = hidden_states.reshape(t, h)
    bias = e_score_correction_bias.reshape(N_EXPERTS, 1)
    bias_b = jnp.broadcast_to(bias, (N_EXPERTS, LANES)).reshape(-1)
    ts = t // N_SLICES
    idx_parts = []
    wgt_parts = []
    for s in range(N_SLICES):
        sfc_s = _scores(hs, weight, bias, ts, s)
        idx_s, wgt_s = _routing_sc(sfc_s, bias_b, ts)
        idx_parts.append(idx_s)
        wgt_parts.append(wgt_s)
    idx_flat = jnp.concatenate(idx_parts)
    wgt_flat = jnp.concatenate(wgt_parts)
    return idx_flat.reshape(t, TOP_K), wgt_flat.reshape(t, TOP_K)
